# baseline (speedup 1.0000x reference)
import jax
import jax.numpy as jnp
from jax.experimental import pallas as pl
from jax.experimental.pallas import tpu as pltpu

N_DOCS = 5
Q = 1024
D = 128
K = 100000
BK = 4096
NK = (K + BK - 1) // BK
KPAD = NK * BK
CHUNK = 128
NCH = BK // CHUNK

NEG_INF = float("-inf")
IMAX = jnp.iinfo(jnp.int32).max


def _body(q_ref, k_ref, tv_out, ti_out, tv_ref, ti_ref):
    kb = pl.program_id(0)

    @pl.when(kb == 0)
    def _init():
        tv_ref[...] = jnp.full(tv_ref.shape, NEG_INF, jnp.float32)
        ti_ref[...] = jnp.zeros(ti_ref.shape, jnp.int32)

    s = jax.lax.dot_general(
        q_ref[...], k_ref[...],
        dimension_numbers=(((1,), (1,)), ((), ())),
        preferred_element_type=jnp.float32,
    )

    for r in range(NCH):
        w = s[:, r * CHUNK:(r + 1) * CHUNK]
        wid = kb * NCH + r
        for t in range(N_DOCS):
            tv = tv_ref[t]
            ti = ti_ref[t]
            gt = w > tv
            tv_ref[t] = jnp.maximum(tv, w)
            ti_ref[t] = jnp.where(gt, wid, ti)
            if t < N_DOCS - 1:
                w, wid = jnp.minimum(tv, w), jnp.where(gt, ti, wid)

    @pl.when(kb == NK - 1)
    def _flush():
        tv_out[...] = tv_ref[...]
        ti_out[...] = ti_ref[...]


def _merge_body(tv_ref, ti_ref, out_v_ref, out_i_ref):
    cv = jnp.concatenate([tv_ref[t] for t in range(N_DOCS)], axis=1)
    cc = jnp.concatenate([ti_ref[t] for t in range(N_DOCS)], axis=1)
    lane = jax.lax.rem(
        jax.lax.broadcasted_iota(jnp.int32, (Q, N_DOCS * CHUNK), 1), CHUNK)
    ci = cc * CHUNK + lane
    cv = jnp.where(ci >= K, NEG_INF, cv)
    for t in range(N_DOCS):
        m = jnp.max(cv, axis=1, keepdims=True)
        hit = cv == m
        sel = jnp.min(jnp.where(hit, ci, IMAX), axis=1, keepdims=True)
        out_v_ref[:, pl.ds(t, 1)] = m
        out_i_ref[:, pl.ds(t, 1)] = sel
        cv = jnp.where(hit & (ci == sel), NEG_INF, cv)


def kernel(queries, keys):
    keys_p = jnp.pad(keys, ((0, KPAD - K), (0, 0)))
    tv, ti = pl.pallas_call(
        _body,
        grid=(NK,),
        in_specs=[
            pl.BlockSpec((Q, D), lambda k: (0, 0)),
            pl.BlockSpec((BK, D), lambda k: (k, 0)),
        ],
        out_specs=[
            pl.BlockSpec((N_DOCS, Q, CHUNK), lambda k: (0, 0, 0)),
            pl.BlockSpec((N_DOCS, Q, CHUNK), lambda k: (0, 0, 0)),
        ],
        out_shape=[
            jax.ShapeDtypeStruct((N_DOCS, Q, CHUNK), jnp.float32),
            jax.ShapeDtypeStruct((N_DOCS, Q, CHUNK), jnp.int32),
        ],
        scratch_shapes=[
            pltpu.VMEM((N_DOCS, Q, CHUNK), jnp.float32),
            pltpu.VMEM((N_DOCS, Q, CHUNK), jnp.int32),
        ],
        compiler_params=pltpu.CompilerParams(
            dimension_semantics=("arbitrary",),
        ),
    )(queries, keys_p)
    out_v, out_i = pl.pallas_call(
        _merge_body,
        out_shape=[
            jax.ShapeDtypeStruct((Q, N_DOCS), jnp.float32),
            jax.ShapeDtypeStruct((Q, N_DOCS), jnp.int32),
        ],
    )(tv, ti)
    return out_v, out_i


# speedup vs baseline: 1.3790x; 1.3790x over previous
import jax
import jax.numpy as jnp
from jax.experimental import pallas as pl
from jax.experimental.pallas import tpu as pltpu

N_DOCS = 5
NCAP = 3
Q = 1024
D = 128
K = 100000
BK = 4096
NK = (K + BK - 1) // BK
KPAD = NK * BK
CHUNK = 128
NCH = BK // CHUNK

NEG_INF = float("-inf")
IMAX = jnp.iinfo(jnp.int32).max


def _dot(q, k):
    return jax.lax.dot_general(
        q, k, dimension_numbers=(((1,), (1,)), ((), ())),
        preferred_element_type=jnp.float32)


def _sweep3_body(q_ref, k_ref, tv_out, ti_out, v4_out, tv_ref, ti_ref, v4_ref):
    kb = pl.program_id(0)

    @pl.when(kb == 0)
    def _init():
        tv_ref[...] = jnp.full(tv_ref.shape, NEG_INF, jnp.float32)
        ti_ref[...] = jnp.zeros(ti_ref.shape, jnp.int32)
        v4_ref[...] = jnp.full(v4_ref.shape, NEG_INF, jnp.float32)

    s = _dot(q_ref[...], k_ref[...])

    for r in range(NCH):
        w = s[:, r * CHUNK:(r + 1) * CHUNK]
        wid = kb * NCH + r
        for t in range(NCAP):
            tv = tv_ref[t]
            ti = ti_ref[t]
            gt = w > tv
            tv_ref[t] = jnp.maximum(tv, w)
            ti_ref[t] = jnp.where(gt, wid, ti)
            if t < NCAP - 1:
                w, wid = jnp.minimum(tv, w), jnp.where(gt, ti, wid)
            else:
                w = jnp.minimum(tv, w)
        v4_ref[...] = jnp.maximum(v4_ref[...], w)

    @pl.when(kb == NK - 1)
    def _flush():
        tv_out[...] = tv_ref[...]
        ti_out[...] = ti_ref[...]
        v4_out[...] = v4_ref[...]


def _merge3_body(tv_ref, ti_ref, v4_ref, out_v_ref, out_i_ref, flag_ref):
    cv = jnp.concatenate([tv_ref[t] for t in range(NCAP)], axis=1)
    cc = jnp.concatenate([ti_ref[t] for t in range(NCAP)], axis=1)
    lane = jax.lax.rem(
        jax.lax.broadcasted_iota(jnp.int32, (Q, NCAP * CHUNK), 1), CHUNK)
    ci = cc * CHUNK + lane
    cv = jnp.where(ci >= K, NEG_INF, cv)
    x5 = None
    for t in range(N_DOCS):
        m = jnp.max(cv, axis=1, keepdims=True)
        hit = cv == m
        sel = jnp.min(jnp.where(hit, ci, IMAX), axis=1, keepdims=True)
        out_v_ref[:, pl.ds(t, 1)] = m
        out_i_ref[:, pl.ds(t, 1)] = sel
        cv = jnp.where(hit & (ci == sel), NEG_INF, cv)
        x5 = m
    mv4 = jnp.max(v4_ref[...], axis=1, keepdims=True)
    n_suspect = jnp.sum((mv4 >= x5).astype(jnp.int32))
    flag_ref[...] = jnp.broadcast_to(n_suspect, flag_ref.shape)


def _sweep5_body(q_ref, k_ref, tv_out, ti_out, tv_ref, ti_ref):
    kb = pl.program_id(0)

    @pl.when(kb == 0)
    def _init():
        tv_ref[...] = jnp.full(tv_ref.shape, NEG_INF, jnp.float32)
        ti_ref[...] = jnp.zeros(ti_ref.shape, jnp.int32)

    s = _dot(q_ref[...], k_ref[...])

    for r in range(NCH):
        w = s[:, r * CHUNK:(r + 1) * CHUNK]
        wid = kb * NCH + r
        for t in range(N_DOCS):
            tv = tv_ref[t]
            ti = ti_ref[t]
            gt = w > tv
            tv_ref[t] = jnp.maximum(tv, w)
            ti_ref[t] = jnp.where(gt, wid, ti)
            if t < N_DOCS - 1:
                w, wid = jnp.minimum(tv, w), jnp.where(gt, ti, wid)

    @pl.when(kb == NK - 1)
    def _flush():
        tv_out[...] = tv_ref[...]
        ti_out[...] = ti_ref[...]


def _merge5_body(tv_ref, ti_ref, out_v_ref, out_i_ref):
    cv = jnp.concatenate([tv_ref[t] for t in range(N_DOCS)], axis=1)
    cc = jnp.concatenate([ti_ref[t] for t in range(N_DOCS)], axis=1)
    lane = jax.lax.rem(
        jax.lax.broadcasted_iota(jnp.int32, (Q, N_DOCS * CHUNK), 1), CHUNK)
    ci = cc * CHUNK + lane
    cv = jnp.where(ci >= K, NEG_INF, cv)
    for t in range(N_DOCS):
        m = jnp.max(cv, axis=1, keepdims=True)
        hit = cv == m
        sel = jnp.min(jnp.where(hit, ci, IMAX), axis=1, keepdims=True)
        out_v_ref[:, pl.ds(t, 1)] = m
        out_i_ref[:, pl.ds(t, 1)] = sel
        cv = jnp.where(hit & (ci == sel), NEG_INF, cv)


def _run_sweep(body, depth, queries, keys_p):
    return pl.pallas_call(
        body,
        grid=(NK,),
        in_specs=[
            pl.BlockSpec((Q, D), lambda k: (0, 0)),
            pl.BlockSpec((BK, D), lambda k: (k, 0)),
        ],
        out_specs=[
            pl.BlockSpec((depth, Q, CHUNK), lambda k: (0, 0, 0)),
            pl.BlockSpec((depth, Q, CHUNK), lambda k: (0, 0, 0)),
        ] + ([pl.BlockSpec((Q, CHUNK), lambda k: (0, 0))]
             if depth == NCAP else []),
        out_shape=[
            jax.ShapeDtypeStruct((depth, Q, CHUNK), jnp.float32),
            jax.ShapeDtypeStruct((depth, Q, CHUNK), jnp.int32),
        ] + ([jax.ShapeDtypeStruct((Q, CHUNK), jnp.float32)]
             if depth == NCAP else []),
        scratch_shapes=[
            pltpu.VMEM((depth, Q, CHUNK), jnp.float32),
            pltpu.VMEM((depth, Q, CHUNK), jnp.int32),
        ] + ([pltpu.VMEM((Q, CHUNK), jnp.float32)] if depth == NCAP else []),
        compiler_params=pltpu.CompilerParams(
            dimension_semantics=("arbitrary",),
        ),
    )(queries, keys_p)


def kernel(queries, keys):
    keys_p = jnp.pad(keys, ((0, KPAD - K), (0, 0)))

    tv, ti, v4 = _run_sweep(_sweep3_body, NCAP, queries, keys_p)
    out_v, out_i, flag = pl.pallas_call(
        _merge3_body,
        out_shape=[
            jax.ShapeDtypeStruct((Q, N_DOCS), jnp.float32),
            jax.ShapeDtypeStruct((Q, N_DOCS), jnp.int32),
            jax.ShapeDtypeStruct((8, 128), jnp.int32),
        ],
    )(tv, ti, v4)

    def _slow_path(_):
        tv5, ti5 = _run_sweep(_sweep5_body, N_DOCS, queries, keys_p)
        return pl.pallas_call(
            _merge5_body,
            out_shape=[
                jax.ShapeDtypeStruct((Q, N_DOCS), jnp.float32),
                jax.ShapeDtypeStruct((Q, N_DOCS), jnp.int32),
            ],
        )(tv5, ti5)

    return jax.lax.cond(
        flag[0, 0] > 0, _slow_path, lambda _: (out_v, out_i), None)
